# baseline (speedup 1.0000x reference)
import jax
import jax.numpy as jnp
from jax import lax
from jax.experimental import pallas as pl
from jax.experimental.pallas import tpu as pltpu
from jax.experimental.pallas import tpu_sc as plsc

EMB = 64
B, LSEQ, T = 1024, 50, 20
NC, NS, LANES = 2, 16, 16
NW = NC * NS
NE = EMB // LANES
ROWS = B * LSEQ
G = 40
BLKS = ROWS // G
BPW = BLKS // NW
NPAIR = BPW // 2
IDX_PER_BLK = G * T
IDX_CHUNK = 80
NSUB = IDX_PER_BLK // IDX_CHUNK


def _sc_body(seq_hbm, w_hbm, out_hbm,
             idx0, idx1, rows0, rows1, out0, out1,
             isem0, isem1, gsem0, gsem1, osem0, osem1):
    wid = lax.axis_index("s") * NC + lax.axis_index("c")

    bufs = ((idx0, rows0, out0, isem0, gsem0, osem0),
            (idx1, rows1, out1, isem1, gsem1, osem1))

    def stage_idx(blk, buf):
        idx_v, _, _, isem, _, _ = bufs[buf]
        pltpu.async_copy(seq_hbm.at[blk], idx_v, isem)

    def fire(blk, buf):
        idx_v, rows_v, _, isem, gsem, _ = bufs[buf]
        pltpu.make_async_copy(seq_hbm.at[blk], idx_v, isem).wait()
        for j in range(NSUB):
            pltpu.async_copy(
                w_hbm.at[idx_v.at[j]],
                rows_v.at[pl.ds(j * IDX_CHUNK, IDX_CHUNK)],
                gsem,
            )

    def drain_rows(buf):
        _, rows_v, _, _, gsem, _ = bufs[buf]
        pltpu.make_async_copy(
            w_hbm.at[pl.ds(0, IDX_PER_BLK)], rows_v, gsem).wait()

    def compute(blk, buf, first):
        idx_v, rows_v, out_v, _, _, osem = bufs[buf]

        @pl.when(jnp.logical_not(first))
        def _():
            pltpu.make_async_copy(
                out_v, out_hbm.at[pl.ds((blk - 2) * G, G)], osem).wait()

        dnums = lax.GatherDimensionNumbers(
            offset_dims=(), collapsed_slice_dims=(0,), start_index_map=(0,))

        @plsc.parallel_loop(0, G, 1, unroll=2)
        def group(g):
            base = g * T
            v1 = idx_v[g >> 2, pl.ds((g & 3) * T, LANES)]
            v2 = idx_v[g >> 2, pl.ds((g & 3) * T + (T - LANES), LANES)]
            iot = lax.iota(jnp.int32, LANES)
            z = jnp.where(v1 == 0, 1, 0) + jnp.where(
                jnp.logical_and(v2 == 0, iot >= 2 * LANES - T), 1, 0)
            for sh in (8, 4, 2, 1):
                perm = ((iot + sh) & (LANES - 1)).reshape(LANES, 1)
                z = z + lax.gather(
                    z, perm, dnums, (1,),
                    mode=lax.GatherScatterMode.PROMISE_IN_BOUNDS)
            cntf = (T - z).astype(jnp.float32)
            inv = jnp.where(z == T, 0.0, 1.0 / cntf)
            s = [jnp.zeros((LANES,), jnp.float32) for _ in range(NE)]
            for t in range(T):
                for e in range(NE):
                    s[e] = s[e] + rows_v[base + t, pl.ds(e * LANES, LANES)]
            for e in range(NE):
                out_v[g, pl.ds(e * LANES, LANES)] = s[e] * inv

        pltpu.async_copy(out_v, out_hbm.at[pl.ds(blk * G, G)], osem)

    base_blk = wid * BPW
    stage_idx(base_blk, 0)
    stage_idx(base_blk + 1, 1)
    fire(base_blk, 0)

    def outer(io, carry):
        blk = base_blk + 2 * io

        drain_rows(0)
        fire(blk + 1, 1)
        compute(blk, 0, first=io == 0)

        @pl.when(io < NPAIR - 1)
        def _():
            stage_idx(blk + 2, 0)

        drain_rows(1)
        compute(blk + 1, 1, first=io == 0)

        @pl.when(io < NPAIR - 1)
        def _():
            stage_idx(blk + 3, 1)
            fire(blk + 2, 0)

        return carry

    lax.fori_loop(0, NPAIR, outer, 0)
    for buf in range(2):
        _, _, out_v, _, _, osem = bufs[buf]
        last = base_blk + BPW - 2 + buf
        pltpu.make_async_copy(
            out_v, out_hbm.at[pl.ds(last * G, G)], osem).wait()


def kernel(seq, W):
    w_full = jnp.concatenate([jnp.zeros((1, EMB), W.dtype), W], axis=0)
    seq3 = seq.reshape(BLKS, NSUB, IDX_CHUNK)
    mesh = plsc.VectorSubcoreMesh(core_axis_name="c", subcore_axis_name="s")
    out = pl.kernel(
        _sc_body,
        mesh=mesh,
        compiler_params=pltpu.CompilerParams(use_tc_tiling_on_sc=False),
        out_type=jax.ShapeDtypeStruct((ROWS, EMB), jnp.float32),
        scratch_types=[
            pltpu.VMEM((NSUB, IDX_CHUNK), jnp.int32),
            pltpu.VMEM((NSUB, IDX_CHUNK), jnp.int32),
            pltpu.VMEM((IDX_PER_BLK, EMB), jnp.float32),
            pltpu.VMEM((IDX_PER_BLK, EMB), jnp.float32),
            pltpu.VMEM((G, EMB), jnp.float32),
            pltpu.VMEM((G, EMB), jnp.float32),
            pltpu.SemaphoreType.DMA,
            pltpu.SemaphoreType.DMA,
            pltpu.SemaphoreType.DMA,
            pltpu.SemaphoreType.DMA,
            pltpu.SemaphoreType.DMA,
            pltpu.SemaphoreType.DMA,
        ],
    )(seq3, w_full)
    return out.reshape(B, LSEQ, EMB)


# speedup vs baseline: 1.0654x; 1.0654x over previous
import jax
import jax.numpy as jnp
from jax import lax
from jax.experimental import pallas as pl
from jax.experimental.pallas import tpu as pltpu
from jax.experimental.pallas import tpu_sc as plsc

EMB = 64
B, LSEQ, T = 1024, 50, 20
NC, NS, LANES = 2, 16, 16
NW = NC * NS
NE = EMB // LANES
ROWS = B * LSEQ
G = 40
BLKS = ROWS // G
BPW = BLKS // NW
NPAIR = BPW // 2
IDX_PER_BLK = G * T
IDX_CHUNK = 80
NSUB = IDX_PER_BLK // IDX_CHUNK


def _sc_body(seq_hbm, w_hbm, out_hbm,
             idx0, idx1, rows0, rows1, out0, out1,
             isem0, isem1, gsem0, gsem1, osem0, osem1):
    wid = lax.axis_index("s") * NC + lax.axis_index("c")

    bufs = ((idx0, rows0, out0, isem0, gsem0, osem0),
            (idx1, rows1, out1, isem1, gsem1, osem1))

    def stage_idx(blk, buf):
        idx_v, _, _, isem, _, _ = bufs[buf]
        pltpu.async_copy(seq_hbm.at[blk], idx_v, isem)

    def fire(blk, buf):
        idx_v, rows_v, _, isem, gsem, _ = bufs[buf]
        pltpu.make_async_copy(seq_hbm.at[blk], idx_v, isem).wait()
        for j in range(NSUB):
            pltpu.async_copy(
                w_hbm.at[idx_v.at[j]],
                rows_v.at[pl.ds(j * IDX_CHUNK, IDX_CHUNK)],
                gsem,
            )

    def drain_rows(buf):
        _, rows_v, _, _, gsem, _ = bufs[buf]
        pltpu.make_async_copy(
            w_hbm.at[pl.ds(0, IDX_PER_BLK)], rows_v, gsem).wait()

    def compute(blk, buf, first):
        _, rows_v, out_v, _, _, osem = bufs[buf]

        @pl.when(jnp.logical_not(first))
        def _():
            pltpu.make_async_copy(
                out_v, out_hbm.at[pl.ds((blk - 2) * G, G)], osem).wait()

        @plsc.parallel_loop(0, G, 1, unroll=2)
        def group(g):
            base = g * T
            s = [jnp.zeros((LANES,), jnp.float32) for _ in range(NE)]
            c = [jnp.zeros((LANES,), jnp.int32) for _ in range(NE)]
            for t in range(T):
                for e in range(NE):
                    r = rows_v[base + t, pl.ds(e * LANES, LANES)]
                    s[e] = s[e] + r
                    bb = lax.bitcast_convert_type(r, jnp.int32)
                    c[e] = c[e] + jnp.where(bb != 0, 1, 0)
            for e in range(NE):
                cf = c[e].astype(jnp.float32)
                out_v[g, pl.ds(e * LANES, LANES)] = jnp.where(
                    c[e] == 0, 0.0, s[e] / cf)

        pltpu.async_copy(out_v, out_hbm.at[pl.ds(blk * G, G)], osem)

    base_blk = wid * BPW
    stage_idx(base_blk, 0)
    stage_idx(base_blk + 1, 1)
    fire(base_blk, 0)

    def outer(io, carry):
        blk = base_blk + 2 * io

        drain_rows(0)

        @pl.when(io < NPAIR - 1)
        def _():
            stage_idx(blk + 2, 0)

        fire(blk + 1, 1)
        compute(blk, 0, first=io == 0)

        drain_rows(1)

        @pl.when(io < NPAIR - 1)
        def _():
            stage_idx(blk + 3, 1)
            fire(blk + 2, 0)

        compute(blk + 1, 1, first=io == 0)
        return carry

    lax.fori_loop(0, NPAIR, outer, 0)
    for buf in range(2):
        _, _, out_v, _, _, osem = bufs[buf]
        last = base_blk + BPW - 2 + buf
        pltpu.make_async_copy(
            out_v, out_hbm.at[pl.ds(last * G, G)], osem).wait()


def kernel(seq, W):
    w_full = jnp.concatenate([jnp.zeros((1, EMB), W.dtype), W], axis=0)
    seq3 = seq.reshape(BLKS, NSUB, IDX_CHUNK)
    mesh = plsc.VectorSubcoreMesh(core_axis_name="c", subcore_axis_name="s")
    out = pl.kernel(
        _sc_body,
        mesh=mesh,
        compiler_params=pltpu.CompilerParams(use_tc_tiling_on_sc=False),
        out_type=jax.ShapeDtypeStruct((ROWS, EMB), jnp.float32),
        scratch_types=[
            pltpu.VMEM((NSUB, IDX_CHUNK), jnp.int32),
            pltpu.VMEM((NSUB, IDX_CHUNK), jnp.int32),
            pltpu.VMEM((IDX_PER_BLK, EMB), jnp.float32),
            pltpu.VMEM((IDX_PER_BLK, EMB), jnp.float32),
            pltpu.VMEM((G, EMB), jnp.float32),
            pltpu.VMEM((G, EMB), jnp.float32),
            pltpu.SemaphoreType.DMA,
            pltpu.SemaphoreType.DMA,
            pltpu.SemaphoreType.DMA,
            pltpu.SemaphoreType.DMA,
            pltpu.SemaphoreType.DMA,
            pltpu.SemaphoreType.DMA,
        ],
    )(seq3, w_full)
    return out.reshape(B, LSEQ, EMB)
